# baseline (speedup 1.0000x reference)
import functools

import jax
import jax.numpy as jnp
from jax import lax
from jax.experimental import pallas as pl
from jax.experimental.pallas import tpu as pltpu
from jax.experimental.pallas import tpu_sc as plsc

_NUM_CORES = 2
_NUM_SUBCORES = 16
_NUM_WORKERS = _NUM_CORES * _NUM_SUBCORES
_IDX_CHUNK = 128
_PACK = 4


def _sc_gather_blocks(theta_packed, bidx2d, batch):
    b_per_w = batch // _NUM_WORKERS
    chunks = b_per_w // _IDX_CHUNK
    width = _PACK * 32
    mesh = plsc.VectorSubcoreMesh(core_axis_name="c", subcore_axis_name="s")

    @functools.partial(
        pl.kernel,
        mesh=mesh,
        out_type=jax.ShapeDtypeStruct((batch, width), jnp.float32),
        scratch_types=[
            pltpu.VMEM((chunks, _IDX_CHUNK), jnp.int32),
            pltpu.VMEM((b_per_w, width), jnp.float32),
            pltpu.SemaphoreType.DMA,
        ],
    )
    def gather_kernel(theta_hbm, idx_hbm, out_hbm, idx_v, rows_v, sem):
        wid = lax.axis_index("s") * _NUM_CORES + lax.axis_index("c")
        base = wid * b_per_w
        pltpu.sync_copy(idx_hbm.at[pl.ds(wid * chunks, chunks)], idx_v)
        copies = []
        for j in range(chunks):
            copies.append(
                pltpu.async_copy(
                    theta_hbm.at[idx_v.at[j]],
                    rows_v.at[pl.ds(j * _IDX_CHUNK, _IDX_CHUNK)],
                    sem,
                )
            )
        for c in copies:
            c.wait()
        pltpu.sync_copy(rows_v, out_hbm.at[pl.ds(base, b_per_w)])

    return gather_kernel(theta_packed, bidx2d)


def _tc_select_matmul_logsoftmax(theta4, sub, alpha_t, batch, num_items):
    blk = 1024
    width = _PACK * 32

    def body(theta_ref, sub_ref, alpha_ref, out_ref):
        th = theta_ref[...]
        group = lax.broadcasted_iota(jnp.int32, (1, width), 1) // 32
        keep = group == sub_ref[...]
        th = jnp.where(keep, th, 0.0)
        u = lax.dot_general(
            th, alpha_ref[...], (((1,), (0,)), ((), ())),
            preferred_element_type=jnp.float32,
        )
        m = jnp.max(u, axis=-1, keepdims=True)
        e = jnp.exp(u - m)
        s = jnp.sum(e, axis=-1, keepdims=True)
        out_ref[...] = u - m - jnp.log(s)

    return pl.pallas_call(
        body,
        grid=(batch // blk,),
        in_specs=[
            pl.BlockSpec((blk, width), lambda i: (i, 0)),
            pl.BlockSpec((blk, 1), lambda i: (i, 0)),
            pl.BlockSpec((width, num_items), lambda i: (0, 0)),
        ],
        out_specs=pl.BlockSpec((blk, num_items), lambda i: (i, 0)),
        out_shape=jax.ShapeDtypeStruct((batch, num_items), jnp.float32),
    )(theta4, sub, alpha_t)


def kernel(user_index, theta_user, alpha_item):
    batch = user_index.shape[0]
    num_users, dim = theta_user.shape
    num_items = alpha_item.shape[0]
    idx = user_index.astype(jnp.int32)
    bidx2d = (idx // _PACK).reshape(batch // _IDX_CHUNK, _IDX_CHUNK)
    sub = (idx % _PACK).reshape(batch, 1)
    theta_packed = theta_user.reshape(num_users // _PACK, _PACK * dim)
    alpha_t = jnp.tile(alpha_item.T, (_PACK, 1))
    theta4 = _sc_gather_blocks(theta_packed, bidx2d, batch)
    return _tc_select_matmul_logsoftmax(theta4, sub, alpha_t, batch, num_items)
